# baseline (speedup 1.0000x reference)
import jax
import jax.numpy as jnp
from jax import lax
from jax.experimental import pallas as pl
from jax.experimental.pallas import tpu as pltpu
from jax.experimental.pallas import tpu_sc as plsc

_MARGIN = 1.0
_DIM = 64
_NV = _DIM // 16


def _sc_partials(ent_emb, rel_emb, h_idx, r_idx, t_idx, hn_idx, tn_idx):
    B = h_idx.shape[0]
    info = plsc.get_sparse_core_info()
    nc, ns = info.num_cores, info.num_subcores
    nw = nc * ns
    per_w = B // nw
    chunk = 128 if per_w % 128 == 0 else per_w
    nchunk = per_w // chunk
    mesh = plsc.VectorSubcoreMesh(core_axis_name="c", subcore_axis_name="s")

    def body(ent, rel, hi_h, ri_h, ti_h, hni_h, tni_h, pos_out, neg_out,
             hi, ri, ti, hni, tni, hv, rv, tv, hnv, tnv, opos, oneg, sem):
        wid = lax.axis_index("s") * nc + lax.axis_index("c")

        def do_chunk(ci, carry):
            base = wid * per_w + ci * chunk
            pltpu.sync_copy(hi_h.at[pl.ds(base, chunk)], hi)
            pltpu.sync_copy(ri_h.at[pl.ds(base, chunk)], ri)
            pltpu.sync_copy(ti_h.at[pl.ds(base, chunk)], ti)
            pltpu.sync_copy(hni_h.at[pl.ds(base, chunk)], hni)
            pltpu.sync_copy(tni_h.at[pl.ds(base, chunk)], tni)
            c1 = pltpu.async_copy(ent.at[hi], hv, sem)
            c2 = pltpu.async_copy(rel.at[ri], rv, sem)
            c3 = pltpu.async_copy(ent.at[ti], tv, sem)
            c4 = pltpu.async_copy(ent.at[hni], hnv, sem)
            c5 = pltpu.async_copy(ent.at[tni], tnv, sem)
            c1.wait()
            c2.wait()
            c3.wait()
            c4.wait()
            c5.wait()

            def do_sample(i, carry2):
                accp = None
                accn = None
                for j in range(_NV):
                    sl = pl.ds(j * 16, 16)
                    rj = rv[i, sl]
                    d = hv[i, sl] + rj - tv[i, sl]
                    dn = hnv[i, sl] + rj - tnv[i, sl]
                    accp = d * d if accp is None else accp + d * d
                    accn = dn * dn if accn is None else accn + dn * dn
                opos[i, :] = accp
                oneg[i, :] = accn
                return carry2

            lax.fori_loop(0, chunk, do_sample, 0, unroll=2)
            pltpu.sync_copy(opos, pos_out.at[pl.ds(base, chunk)])
            pltpu.sync_copy(oneg, neg_out.at[pl.ds(base, chunk)])
            return carry

        lax.fori_loop(0, nchunk, do_chunk, 0)

    f = pl.kernel(
        body,
        out_type=(
            jax.ShapeDtypeStruct((B, 16), jnp.float32),
            jax.ShapeDtypeStruct((B, 16), jnp.float32),
        ),
        mesh=mesh,
        compiler_params=pltpu.CompilerParams(use_tc_tiling_on_sc=False),
        scratch_types=(
            [pltpu.VMEM((chunk,), jnp.int32) for _ in range(5)]
            + [pltpu.VMEM((chunk, _DIM), jnp.float32) for _ in range(5)]
            + [pltpu.VMEM((chunk, 16), jnp.float32) for _ in range(2)]
            + [pltpu.SemaphoreType.DMA]
        ),
    )
    return f(ent_emb, rel_emb, h_idx, r_idx, t_idx, hn_idx, tn_idx)


def _tc_loss(pos_part, neg_part):
    B = pos_part.shape[0]
    p2 = pos_part.reshape(B * 16 // 128, 128)
    n2 = neg_part.reshape(B * 16 // 128, 128)

    def body(p_ref, n_ref, o_ref):
        row = lax.broadcasted_iota(jnp.int32, (128, 8), 0)
        col = lax.broadcasted_iota(jnp.int32, (128, 8), 1)
        m = jnp.where(row // 16 == col, 1.0, 0.0).astype(jnp.float32)
        ps = jnp.dot(p_ref[...], m, preferred_element_type=jnp.float32)
        ns = jnp.dot(n_ref[...], m, preferred_element_type=jnp.float32)
        v = jnp.maximum(_MARGIN + jnp.sqrt(ps) - jnp.sqrt(ns), 0.0)
        o_ref[0, 0] = jnp.sum(v) * (1.0 / 4096.0)

    out = pl.pallas_call(
        body,
        out_shape=jax.ShapeDtypeStruct((1, 1), jnp.float32),
        out_specs=pl.BlockSpec(memory_space=pltpu.SMEM),
    )(p2, n2)
    return out[0, 0]


def kernel(ent_emb, rel_emb, h_idx, r_idx, t_idx, h_neg_idx, t_neg_idx):
    pos_part, neg_part = _sc_partials(
        ent_emb, rel_emb, h_idx, r_idx, t_idx, h_neg_idx, t_neg_idx
    )
    return _tc_loss(pos_part, neg_part)


# speedup vs baseline: 1.6354x; 1.6354x over previous
import jax
import jax.numpy as jnp
from jax import lax
from jax.experimental import pallas as pl
from jax.experimental.pallas import tpu as pltpu
from jax.experimental.pallas import tpu_sc as plsc

_MARGIN = 1.0
_DIM = 64
_NV = _DIM // 16
_CHUNK = 128


def _sc_partials(ent_emb, rel_emb, h_idx, r_idx, t_idx, hn_idx, tn_idx):
    B = h_idx.shape[0]
    R = rel_emb.shape[0]
    info = plsc.get_sparse_core_info()
    nc, ns = info.num_cores, info.num_subcores
    nw = nc * ns
    per_w = B // nw
    chunk = _CHUNK if per_w % _CHUNK == 0 else per_w
    nchunk = per_w // chunk
    ngrp = chunk // 16
    orow = chunk * 16 // 128
    mesh = plsc.VectorSubcoreMesh(core_axis_name="c", subcore_axis_name="s")

    def body(ent, rel, hi_h, ri_h, ti_h, hni_h, tni_h, pos_out, neg_out,
             hi, ri, ti, hni, tni, hv, rv, tv, hnv, tnv, opos, oneg, sem):
        wid = lax.axis_index("s") * nc + lax.axis_index("c")

        def do_chunk(ci, carry):
            base = pl.multiple_of(wid * per_w + ci * chunk, chunk)
            pltpu.sync_copy(hi_h.at[pl.ds(base, chunk)], hi)
            pltpu.sync_copy(ri_h.at[pl.ds(base, chunk)], ri)
            pltpu.sync_copy(ti_h.at[pl.ds(base, chunk)], ti)
            pltpu.sync_copy(hni_h.at[pl.ds(base, chunk)], hni)
            pltpu.sync_copy(tni_h.at[pl.ds(base, chunk)], tni)

            def fire(g, c2):
                hvec = hi[pl.ds(g * 16, 16)]
                rvec = ri[pl.ds(g * 16, 16)]
                tvec = ti[pl.ds(g * 16, 16)]
                hnvec = hni[pl.ds(g * 16, 16)]
                tnvec = tni[pl.ds(g * 16, 16)]
                for j in range(16):
                    dst = g * 16 + j
                    pltpu.async_copy(
                        ent.at[pl.ds(hvec[j], 1), :],
                        hv.at[pl.ds(dst, 1), :], sem)
                    pltpu.async_copy(
                        rel.at[pl.ds(rvec[j], 1), :],
                        rv.at[pl.ds(dst, 1), :], sem)
                    pltpu.async_copy(
                        ent.at[pl.ds(tvec[j], 1), :],
                        tv.at[pl.ds(dst, 1), :], sem)
                    pltpu.async_copy(
                        ent.at[pl.ds(hnvec[j], 1), :],
                        hnv.at[pl.ds(dst, 1), :], sem)
                    pltpu.async_copy(
                        ent.at[pl.ds(tnvec[j], 1), :],
                        tnv.at[pl.ds(dst, 1), :], sem)
                return c2

            lax.fori_loop(0, ngrp, fire, 0)
            pltpu.make_async_copy(ent.at[pl.ds(0, chunk), :], hv, sem).wait()
            pltpu.make_async_copy(ent.at[pl.ds(0, chunk), :], rv, sem).wait()
            pltpu.make_async_copy(ent.at[pl.ds(0, chunk), :], tv, sem).wait()
            pltpu.make_async_copy(ent.at[pl.ds(0, chunk), :], hnv, sem).wait()
            pltpu.make_async_copy(ent.at[pl.ds(0, chunk), :], tnv, sem).wait()

            def compute(g, c2):
                for j in range(16):
                    i = g * 16 + j
                    accp = None
                    accn = None
                    for k in range(_NV):
                        sl = pl.ds(k * 16, 16)
                        rk = rv[i, sl]
                        d = hv[i, sl] + rk - tv[i, sl]
                        dn = hnv[i, sl] + rk - tnv[i, sl]
                        accp = d * d if accp is None else accp + d * d
                        accn = dn * dn if accn is None else accn + dn * dn
                    opos[2 * g + j // 8, pl.ds((j % 8) * 16, 16)] = accp
                    oneg[2 * g + j // 8, pl.ds((j % 8) * 16, 16)] = accn
                return c2

            lax.fori_loop(0, ngrp, compute, 0)
            row_base = pl.multiple_of(base * 16 // 128, orow)
            pltpu.sync_copy(opos, pos_out.at[pl.ds(row_base, orow), :])
            pltpu.sync_copy(oneg, neg_out.at[pl.ds(row_base, orow), :])
            return carry

        lax.fori_loop(0, nchunk, do_chunk, 0)

    f = pl.kernel(
        body,
        out_type=(
            jax.ShapeDtypeStruct((B * 16 // 128, 128), jnp.float32),
            jax.ShapeDtypeStruct((B * 16 // 128, 128), jnp.float32),
        ),
        mesh=mesh,
        scratch_types=(
            [pltpu.VMEM((chunk,), jnp.int32) for _ in range(5)]
            + [pltpu.VMEM((chunk, _DIM), jnp.float32) for _ in range(5)]
            + [pltpu.VMEM((orow, 128), jnp.float32) for _ in range(2)]
            + [pltpu.SemaphoreType.DMA]
        ),
    )
    return f(ent_emb, rel_emb, h_idx, r_idx, t_idx, hn_idx, tn_idx)


def _tc_loss(pos_part, neg_part):
    def body(p_ref, n_ref, o_ref):
        row = lax.broadcasted_iota(jnp.int32, (128, 8), 0)
        col = lax.broadcasted_iota(jnp.int32, (128, 8), 1)
        m = jnp.where(row // 16 == col, 1.0, 0.0).astype(jnp.float32)
        ps = jnp.dot(p_ref[...], m, preferred_element_type=jnp.float32)
        ns = jnp.dot(n_ref[...], m, preferred_element_type=jnp.float32)
        v = jnp.maximum(_MARGIN + jnp.sqrt(ps) - jnp.sqrt(ns), 0.0)
        o_ref[0, 0] = jnp.sum(v) * (1.0 / 4096.0)

    out = pl.pallas_call(
        body,
        out_shape=jax.ShapeDtypeStruct((1, 1), jnp.float32),
        out_specs=pl.BlockSpec(memory_space=pltpu.SMEM),
    )(pos_part, neg_part)
    return out[0, 0]


def kernel(ent_emb, rel_emb, h_idx, r_idx, t_idx, h_neg_idx, t_neg_idx):
    pos_part, neg_part = _sc_partials(
        ent_emb, rel_emb, h_idx, r_idx, t_idx, h_neg_idx, t_neg_idx
    )
    return _tc_loss(pos_part, neg_part)


# speedup vs baseline: 1.6391x; 1.0023x over previous
import jax
import jax.numpy as jnp
from jax import lax
from jax.experimental import pallas as pl
from jax.experimental.pallas import tpu as pltpu
from jax.experimental.pallas import tpu_sc as plsc

_MARGIN = 1.0
_DIM = 64
_NV = _DIM // 16
_CHUNK = 128


def _sc_partials(ent_emb, rel_emb, h_idx, r_idx, t_idx, hn_idx, tn_idx):
    B = h_idx.shape[0]
    R = rel_emb.shape[0]
    info = plsc.get_sparse_core_info()
    nc, ns = info.num_cores, info.num_subcores
    nw = nc * ns
    per_w = B // nw
    chunk = _CHUNK if per_w % _CHUNK == 0 else per_w
    nchunk = per_w // chunk
    ngrp = chunk // 16
    orow = chunk * 16 // 128
    mesh = plsc.VectorSubcoreMesh(core_axis_name="c", subcore_axis_name="s")

    def body(ent, rel, hi_h, ri_h, ti_h, hni_h, tni_h, pos_out, neg_out,
             hi, ri, ti, hni, tni, hv, rv, tv, hnv, tnv, opos, oneg, sem):
        wid = lax.axis_index("s") * nc + lax.axis_index("c")

        def do_chunk(ci, carry):
            base = pl.multiple_of(wid * per_w + ci * chunk, chunk)
            pltpu.sync_copy(hi_h.at[pl.ds(base, chunk)], hi)
            pltpu.sync_copy(ri_h.at[pl.ds(base, chunk)], ri)
            pltpu.sync_copy(ti_h.at[pl.ds(base, chunk)], ti)
            pltpu.sync_copy(hni_h.at[pl.ds(base, chunk)], hni)
            pltpu.sync_copy(tni_h.at[pl.ds(base, chunk)], tni)

            def fire(g, c2):
                hvec = hi[pl.ds(g * 16, 16)]
                rvec = ri[pl.ds(g * 16, 16)]
                tvec = ti[pl.ds(g * 16, 16)]
                hnvec = hni[pl.ds(g * 16, 16)]
                tnvec = tni[pl.ds(g * 16, 16)]
                for j in range(16):
                    dst = g * 16 + j
                    pltpu.async_copy(
                        ent.at[pl.ds(hvec[j], 1), :],
                        hv.at[pl.ds(dst, 1), :], sem)
                    pltpu.async_copy(
                        rel.at[pl.ds(rvec[j], 1), :],
                        rv.at[pl.ds(dst, 1), :], sem)
                    pltpu.async_copy(
                        ent.at[pl.ds(tvec[j], 1), :],
                        tv.at[pl.ds(dst, 1), :], sem)
                    pltpu.async_copy(
                        ent.at[pl.ds(hnvec[j], 1), :],
                        hnv.at[pl.ds(dst, 1), :], sem)
                    pltpu.async_copy(
                        ent.at[pl.ds(tnvec[j], 1), :],
                        tnv.at[pl.ds(dst, 1), :], sem)
                return c2

            lax.fori_loop(0, ngrp, fire, 0)
            pltpu.make_async_copy(ent.at[pl.ds(0, chunk), :], hv, sem).wait()
            pltpu.make_async_copy(ent.at[pl.ds(0, chunk), :], rv, sem).wait()
            pltpu.make_async_copy(ent.at[pl.ds(0, chunk), :], tv, sem).wait()
            pltpu.make_async_copy(ent.at[pl.ds(0, chunk), :], hnv, sem).wait()
            pltpu.make_async_copy(ent.at[pl.ds(0, chunk), :], tnv, sem).wait()

            def compute(g, c2):
                for j in range(16):
                    i = g * 16 + j
                    accp = None
                    accn = None
                    for k in range(_NV):
                        sl = pl.ds(k * 16, 16)
                        rk = rv[i, sl]
                        d = hv[i, sl] + rk - tv[i, sl]
                        dn = hnv[i, sl] + rk - tnv[i, sl]
                        accp = d * d if accp is None else accp + d * d
                        accn = dn * dn if accn is None else accn + dn * dn
                    opos[2 * g + j // 8, pl.ds((j % 8) * 16, 16)] = accp
                    oneg[2 * g + j // 8, pl.ds((j % 8) * 16, 16)] = accn
                return c2

            lax.fori_loop(0, ngrp, compute, 0)
            row_base = pl.multiple_of(base * 16 // 128, orow)
            pltpu.sync_copy(opos, pos_out.at[pl.ds(row_base, orow), :])
            pltpu.sync_copy(oneg, neg_out.at[pl.ds(row_base, orow), :])
            return carry

        lax.fori_loop(0, nchunk, do_chunk, 0)

    f = pl.kernel(
        body,
        out_type=(
            jax.ShapeDtypeStruct((B * 16 // 128, 128), jnp.float32),
            jax.ShapeDtypeStruct((B * 16 // 128, 128), jnp.float32),
        ),
        mesh=mesh,
        compiler_params=pltpu.CompilerParams(use_tc_tiling_on_sc=True),
        scratch_types=(
            [pltpu.VMEM((chunk,), jnp.int32) for _ in range(5)]
            + [pltpu.VMEM((chunk, _DIM), jnp.float32) for _ in range(5)]
            + [pltpu.VMEM((orow, 128), jnp.float32) for _ in range(2)]
            + [pltpu.SemaphoreType.DMA]
        ),
    )
    return f(ent_emb, rel_emb, h_idx, r_idx, t_idx, hn_idx, tn_idx)


def _tc_loss(pos_part, neg_part):
    def body(p_ref, n_ref, o_ref):
        row = lax.broadcasted_iota(jnp.int32, (128, 8), 0)
        col = lax.broadcasted_iota(jnp.int32, (128, 8), 1)
        m = jnp.where(row // 16 == col, 1.0, 0.0).astype(jnp.float32)
        ps = jnp.dot(p_ref[...], m, preferred_element_type=jnp.float32)
        ns = jnp.dot(n_ref[...], m, preferred_element_type=jnp.float32)
        v = jnp.maximum(_MARGIN + jnp.sqrt(ps) - jnp.sqrt(ns), 0.0)
        o_ref[0, 0] = jnp.sum(v) * (1.0 / 4096.0)

    out = pl.pallas_call(
        body,
        out_shape=jax.ShapeDtypeStruct((1, 1), jnp.float32),
        out_specs=pl.BlockSpec(memory_space=pltpu.SMEM),
    )(pos_part, neg_part)
    return out[0, 0]


def kernel(ent_emb, rel_emb, h_idx, r_idx, t_idx, h_neg_idx, t_neg_idx):
    pos_part, neg_part = _sc_partials(
        ent_emb, rel_emb, h_idx, r_idx, t_idx, h_neg_idx, t_neg_idx
    )
    return _tc_loss(pos_part, neg_part)
